# baseline (speedup 1.0000x reference)
import jax
import jax.numpy as jnp
from jax import lax
from jax.experimental import pallas as pl
from jax.experimental.pallas import tpu as pltpu
from jax.experimental.pallas import tpu_sc as plsc

N = 10000
D = 128
H = 4
DH = 128
E = 40000
RD = 5
RT = 4
R = RD + RT

NP = 10240
NT = 16
NW = 32
EPW = E // NW
CHA = 128
NCA = 10
EPWP = NCA * CHA
EXR = EPWP // 8
EPT = 2 * EPWP
CHH = 64
NCH = EPT // CHH
SPT = NP // NT
ZB = SPT // 40
BN = 1024
NB = NP // BN
NLANE = 16

_MESH = plsc.VectorSubcoreMesh(core_axis_name="c", subcore_axis_name="s")


def _feat_body(xs_ref, xd_ref, w_ref, al_ref, ar_ref, hq_ref, hr_ref,
               fs_ref, fd_ref, el_ref, er_ref, q0_ref, rr_ref):
    xs = xs_ref[0]
    xd = xd_ref[0]
    w = w_ref[0]
    fs = jnp.dot(xs, w, preferred_element_type=jnp.float32)
    fd = jnp.dot(xd, w, preferred_element_type=jnp.float32)
    els, ers, q0s, rrs = [], [], [], []
    for h in range(H):
        fsh = fs[:, h * DH:(h + 1) * DH]
        fdh = fd[:, h * DH:(h + 1) * DH]
        fs_ref[0, h] = fsh
        fd_ref[0, h] = fdh
        els.append(jnp.sum(fsh * al_ref[0, h][None, :], axis=1))
        ers.append(jnp.sum(fdh * ar_ref[0, h][None, :], axis=1))
        q0s.append(jnp.sum(fdh * hq_ref[0, h][None, :], axis=1))
        rrs.append(jnp.sum(fdh * hr_ref[0, h][None, :], axis=1))
    el4 = jnp.stack(els, axis=-1)
    er4 = jnp.stack(ers, axis=-1)
    z = jnp.zeros((el4.shape[0], DH - 4 * H), jnp.float32)
    el_ref[0] = jnp.concatenate([el4, el4, el4, el4, z], axis=-1)
    er_ref[0] = jnp.concatenate([er4, er4, er4, er4, z], axis=-1)
    q0_ref[0] = jnp.stack(q0s)
    rr_ref[0] = jnp.stack(rrs)


def _stage_a(xpair, w_all, al_all, ar_all, hq_all, hr_all):
    f_sds = jax.ShapeDtypeStruct((R, H, NP, DH), jnp.float32)
    p_sds = jax.ShapeDtypeStruct((R, NP, DH), jnp.float32)
    v_sds = jax.ShapeDtypeStruct((R, H, NP), jnp.float32)
    wmap = lambda r, i: (r, 0, 0)
    return pl.pallas_call(
        _feat_body,
        grid=(R, NB),
        in_specs=[
            pl.BlockSpec((1, BN, D), lambda r, i: (jnp.where(r < RD, 0, 1), i, 0)),
            pl.BlockSpec((1, BN, D), lambda r, i: (jnp.where(r < RD, 1, 0), i, 0)),
            pl.BlockSpec((1, D, H * DH), wmap),
            pl.BlockSpec((1, H, DH), wmap),
            pl.BlockSpec((1, H, DH), wmap),
            pl.BlockSpec((1, H, DH), wmap),
            pl.BlockSpec((1, H, DH), wmap),
        ],
        out_specs=[
            pl.BlockSpec((1, H, BN, DH), lambda r, i: (r, 0, i, 0)),
            pl.BlockSpec((1, H, BN, DH), lambda r, i: (r, 0, i, 0)),
            pl.BlockSpec((1, BN, DH), lambda r, i: (r, i, 0)),
            pl.BlockSpec((1, BN, DH), lambda r, i: (r, i, 0)),
            pl.BlockSpec((1, H, BN), lambda r, i: (r, 0, i)),
            pl.BlockSpec((1, H, BN), lambda r, i: (r, 0, i)),
        ],
        out_shape=[f_sds, f_sds, p_sds, p_sds, v_sds, v_sds],
    )(xpair, xpair, w_all, al_all, ar_all, hq_all, hr_all)


def _att_body(eltab, ertab, srcp, dstp, ex_hbm,
              src_loc, dst_loc, isrc, idst, rs, rd, ex_buf, sem1, sem2):
    c = lax.axis_index("c")
    s = lax.axis_index("s")
    w = 2 * s + c

    def rel_body(r, carry):
        pltpu.sync_copy(srcp.at[r, w], src_loc)
        pltpu.sync_copy(dstp.at[r, w], dst_loc)
        off = r * NP

        def ib(j, ci):
            row = j // 8
            col = (j % 8) * NLANE
            isrc[row, pl.ds(col, NLANE)] = src_loc[row, pl.ds(col, NLANE)] + off
            idst[row, pl.ds(col, NLANE)] = dst_loc[row, pl.ds(col, NLANE)] + off
            return ci
        lax.fori_loop(0, EPWP // NLANE, ib, 0)

        def chunk(ci, cc):
            d1 = pltpu.async_copy(eltab.at[isrc.at[ci]], rs, sem1)
            d2 = pltpu.async_copy(ertab.at[idst.at[ci]], rd, sem2)
            d1.wait()
            d2.wait()

            def per_edge(j, cs):
                q = ci * CHA + j
                ev = rs[j, pl.ds(0, NLANE)] + rd[j, pl.ds(0, NLANE)]
                ev = jnp.maximum(ev, 0.2 * ev)
                ex = jnp.exp(ev)
                ex = jnp.where(q < EPW, ex, 0.0)
                ex_buf[q // 8, pl.ds((q % 8) * NLANE, NLANE)] = ex
                return cs
            lax.fori_loop(0, CHA, per_edge, 0)
            return cc
        lax.fori_loop(0, NCA, chunk, 0)

        pltpu.sync_copy(ex_buf, ex_hbm.at[r, w])
        return carry
    lax.fori_loop(0, R, rel_body, 0)


def _attention(eltab_flat, ertab_flat, srcp32, dstp32):
    fn = pl.kernel(
        _att_body,
        out_type=jax.ShapeDtypeStruct((R, NW, EXR, 128), jnp.float32),
        mesh=_MESH,
        scratch_types=[
            pltpu.VMEM((NCA, CHA), jnp.int32),
            pltpu.VMEM((NCA, CHA), jnp.int32),
            pltpu.VMEM((NCA, CHA), jnp.int32),
            pltpu.VMEM((NCA, CHA), jnp.int32),
            pltpu.VMEM((CHA, DH), jnp.float32),
            pltpu.VMEM((CHA, DH), jnp.float32),
            pltpu.VMEM((EXR, 128), jnp.float32),
            pltpu.SemaphoreType.DMA,
            pltpu.SemaphoreType.DMA,
        ],
    )
    return fn(eltab_flat, ertab_flat, srcp32, dstp32)


def _expand_body(ex_ref, out_ref):
    x = ex_ref[0]
    ys = []
    for h in range(H):
        groups = []
        for g in range(8):
            col = x[:, :, g * NLANE + h:g * NLANE + h + 1]
            groups.append(jnp.broadcast_to(col, (NW, EXR, NLANE)))
        ys.append(jnp.concatenate(groups, axis=-1))
    y = jnp.stack(ys)
    out_ref[...] = y.reshape(H * NW * EXR, 128)


def _expand(ex32):
    return pl.pallas_call(
        _expand_body,
        grid=(R,),
        in_specs=[pl.BlockSpec((1, NW, EXR, 128), lambda r: (r, 0, 0, 0))],
        out_specs=pl.BlockSpec((H * NW * EXR, 128), lambda r: (r, 0)),
        out_shape=jax.ShapeDtypeStruct((R * H * NW * EXR, 128), jnp.float32),
    )(ex32)


def _hop_body(htab, exsp, srcp, dstp, hsum_hbm,
              acc, rows0, rows1, srows, exc0, exc1,
              src_loc, dst_loc, idx_buf, dst64, zbuf,
              gsem0, gsem1, esem0, esem1, ssem, zsem):
    c = lax.axis_index("c")
    s = lax.axis_index("s")
    zero16 = jnp.zeros((NLANE,), jnp.float32)
    rowsv = (rows0, rows1)
    excv = (exc0, exc1)
    gsemv = (gsem0, gsem1)
    esemv = (esem0, esem1)

    def zz(j, cz):
        zbuf[j // 8, pl.ds((j % 8) * NLANE, NLANE)] = zero16
        return cz
    lax.fori_loop(0, ZB * (DH // NLANE), zz, 0)

    def rel_body(r, carry):
        pltpu.sync_copy(srcp.at[r, s], src_loc)
        pltpu.sync_copy(dstp.at[r, s], dst_loc)

        def db(j, ci):
            row = j // 8
            col = (j % 8) * NLANE
            v = dst_loc[row, pl.ds(col, NLANE)]
            fl = j * NLANE
            dst64[fl // CHH, pl.ds(fl % CHH, NLANE)] = v
            return ci
        lax.fori_loop(0, EPT // NLANE, db, 0)

        for hh in range(2):
            h = 2 * c + hh
            off = (r * H + h) * NP
            row0 = ((r * H + h) * NW + 2 * s) * EXR

            def ib(j, ci):
                row = j // 8
                col = (j % 8) * NLANE
                v = src_loc[row, pl.ds(col, NLANE)] + off
                fl = j * NLANE
                idx_buf[fl // CHH, pl.ds(fl % CHH, NLANE)] = v
                return ci
            lax.fori_loop(0, EPT // NLANE, ib, 0)

            def zc(k, cz):
                pltpu.async_copy(zbuf, acc.at[pl.ds(s * SPT + k * ZB, ZB)], zsem)
                return cz
            lax.fori_loop(0, SPT // ZB, zc, 0)

            def zw(k, cz):
                pltpu.make_async_copy(zbuf, acc.at[pl.ds(s * SPT, ZB)], zsem).wait()
                return cz
            lax.fori_loop(0, SPT // ZB, zw, 0)
            plsc.subcore_barrier()

            for b in range(2):
                pltpu.async_copy(htab.at[idx_buf.at[b]], rowsv[b], gsemv[b])
                pltpu.async_copy(
                    exsp.at[pl.ds(row0 + b * (CHH // 8), CHH // 8)],
                    excv[b], esemv[b])

            def pair(g, cc):
                for b in range(2):
                    ci = g * 2 + b
                    pltpu.make_async_copy(
                        htab.at[idx_buf.at[0]], rowsv[b], gsemv[b]).wait()
                    pltpu.make_async_copy(
                        exsp.at[pl.ds(row0, CHH // 8)], excv[b], esemv[b]).wait()

                    @pl.when(ci >= 1)
                    def _drain():
                        pltpu.make_async_copy(
                            srows, acc.at[dst64.at[0]], ssem).wait()

                    def scale(j, cs):
                        av = excv[b][j // 8, pl.ds((j % 8) * NLANE, NLANE)]
                        for v in range(DH // NLANE):
                            srows[j, pl.ds(v * NLANE, NLANE)] = (
                                rowsv[b][j, pl.ds(v * NLANE, NLANE)] * av)
                        return cs
                    lax.fori_loop(0, CHH, scale, 0)
                    pltpu.async_copy(srows, acc.at[dst64.at[ci]],
                                     ssem, add=True)

                    @pl.when(ci + 2 < NCH)
                    def _prefetch():
                        pltpu.async_copy(htab.at[idx_buf.at[ci + 2]],
                                         rowsv[b], gsemv[b])
                        pltpu.async_copy(
                            exsp.at[pl.ds(row0 + (ci + 2) * (CHH // 8), CHH // 8)],
                            excv[b], esemv[b])
                return cc
            lax.fori_loop(0, NCH // 2, pair, 0)
            pltpu.make_async_copy(srows, acc.at[dst64.at[0]], ssem).wait()
            plsc.subcore_barrier()
            pltpu.sync_copy(acc.at[pl.ds(s * SPT, SPT)],
                            hsum_hbm.at[r, h, pl.ds(s * SPT, SPT)])
            plsc.subcore_barrier()
        return carry
    lax.fori_loop(0, R, rel_body, 0)


def _hop(htab_flat, exsplat, srcp16, dstp16):
    fn = pl.kernel(
        _hop_body,
        out_type=jax.ShapeDtypeStruct((R, H, NP, DH), jnp.float32),
        mesh=_MESH,
        scratch_types=[
            pltpu.VMEM_SHARED((NP, DH), jnp.float32),
            pltpu.VMEM((CHH, DH), jnp.float32),
            pltpu.VMEM((CHH, DH), jnp.float32),
            pltpu.VMEM((CHH, DH), jnp.float32),
            pltpu.VMEM((CHH // 8, 128), jnp.float32),
            pltpu.VMEM((CHH // 8, 128), jnp.float32),
            pltpu.VMEM((EPT // CHA, CHA), jnp.int32),
            pltpu.VMEM((EPT // CHA, CHA), jnp.int32),
            pltpu.VMEM((NCH, CHH), jnp.int32),
            pltpu.VMEM((NCH, CHH), jnp.int32),
            pltpu.VMEM((ZB, DH), jnp.float32),
            pltpu.SemaphoreType.DMA,
            pltpu.SemaphoreType.DMA,
            pltpu.SemaphoreType.DMA,
            pltpu.SemaphoreType.DMA,
            pltpu.SemaphoreType.DMA,
            pltpu.SemaphoreType.DMA,
        ],
    )
    return fn(htab_flat, exsplat, srcp16, dstp16)


def _norm_body(hs_ref, hq_ref, hn_ref, q_ref):
    qs = []
    for h in range(H):
        v = hs_ref[0, h]
        nr = jnp.sqrt(jnp.sum(v * v, axis=-1, keepdims=True))
        y = v / (nr + 1e-30)
        hn_ref[0, h] = y
        qs.append(jnp.sum(y * hq_ref[0, h][None, :], axis=-1))
    q_ref[0] = jnp.stack(qs)


def _norm(hsum, hq_all):
    return pl.pallas_call(
        _norm_body,
        grid=(R, NB),
        in_specs=[
            pl.BlockSpec((1, H, BN, DH), lambda r, i: (r, 0, i, 0)),
            pl.BlockSpec((1, H, DH), lambda r, i: (r, 0, 0)),
        ],
        out_specs=[
            pl.BlockSpec((1, H, BN, DH), lambda r, i: (r, 0, i, 0)),
            pl.BlockSpec((1, H, BN), lambda r, i: (r, 0, i)),
        ],
        out_shape=[
            jax.ShapeDtypeStruct((R, H, NP, DH), jnp.float32),
            jax.ShapeDtypeStruct((R, H, NP), jnp.float32),
        ],
    )(hsum, hq_all)


def _comb_body(wrel_ref, fd_ref, h1_ref, h2_ref, h3_ref,
               q0_ref, q1_ref, q2_ref, q3_ref, rr_ref, out_ref):
    r = pl.program_id(1)
    rr = rr_ref[0]

    def leaky(x):
        return jnp.maximum(x, 0.2 * x)
    l0 = leaky(q0_ref[0] + rr)
    l1 = leaky(q1_ref[0] + rr)
    l2 = leaky(q2_ref[0] + rr)
    l3 = leaky(q3_ref[0] + rr)
    m = jnp.maximum(jnp.maximum(l0, l1), jnp.maximum(l2, l3))
    e0 = jnp.exp(l0 - m)
    e1 = jnp.exp(l1 - m)
    e2 = jnp.exp(l2 - m)
    e3 = jnp.exp(l3 - m)
    den = e0 + e1 + e2 + e3
    w0 = e0 / den + 1.0
    w1 = e1 / den
    w2 = e2 / den
    w3 = e3 / den
    acc = jnp.zeros((BN, DH), jnp.float32)
    for h in range(H):
        acc = acc + (w0[h][:, None] * fd_ref[0, h]
                     + w1[h][:, None] * h1_ref[0, h]
                     + w2[h][:, None] * h2_ref[0, h]
                     + w3[h][:, None] * h3_ref[0, h])
    contrib = (wrel_ref[r] * (1.0 / H)) * acc

    @pl.when(r == 0)
    def _init():
        out_ref[...] = contrib

    @pl.when(r > 0)
    def _acc():
        out_ref[...] += contrib


def _combine(wrel, fd_all, h1, h2, h3, q0, q1, q2, q3, rr, off, rg):
    fmap = lambda i, r: (r + off, 0, i, 0)
    vmap_ = lambda i, r: (r + off, 0, i)
    return pl.pallas_call(
        _comb_body,
        grid=(NB, rg),
        in_specs=[
            pl.BlockSpec(memory_space=pltpu.SMEM),
            pl.BlockSpec((1, H, BN, DH), fmap),
            pl.BlockSpec((1, H, BN, DH), fmap),
            pl.BlockSpec((1, H, BN, DH), fmap),
            pl.BlockSpec((1, H, BN, DH), fmap),
            pl.BlockSpec((1, H, BN), vmap_),
            pl.BlockSpec((1, H, BN), vmap_),
            pl.BlockSpec((1, H, BN), vmap_),
            pl.BlockSpec((1, H, BN), vmap_),
            pl.BlockSpec((1, H, BN), vmap_),
        ],
        out_specs=pl.BlockSpec((BN, DH), lambda i, r: (i, 0)),
        out_shape=jax.ShapeDtypeStruct((NP, DH), jnp.float32),
    )(wrel, fd_all, h1, h2, h3, q0, q1, q2, q3, rr)


def _pad_edges32(a):
    a = a.reshape(R, NW, EPW)
    a = jnp.pad(a, ((0, 0), (0, 0), (0, EPWP - EPW)))
    return a.reshape(R, NW, NCA, CHA).astype(jnp.int32)


def _pad_edges16(a):
    a = a.reshape(R, NW, EPW)
    a = jnp.pad(a, ((0, 0), (0, 0), (0, EPWP - EPW)))
    return a.reshape(R, NT, EPT // CHA, CHA).astype(jnp.int32)


def kernel(x_drug, x_target, edge_index_drug, edge_index_target,
           W_drug, attn_l_drug, attn_r_drug, hop_q_drug, hop_r_drug,
           W_target, attn_l_target, attn_r_target, hop_q_target, hop_r_target,
           Wd, bd, Wt, bt):
    pad = ((0, NP - N), (0, 0))
    xpair = jnp.stack([jnp.pad(x_target, pad), jnp.pad(x_drug, pad)])
    w_all = jnp.concatenate([W_drug, W_target], axis=0)
    al_all = jnp.concatenate([attn_l_drug, attn_l_target], axis=0)
    ar_all = jnp.concatenate([attn_r_drug, attn_r_target], axis=0)
    hq_all = jnp.concatenate([hop_q_drug, hop_q_target], axis=0)
    hr_all = jnp.concatenate([hop_r_drug, hop_r_target], axis=0)
    edge_all = jnp.concatenate([edge_index_drug, edge_index_target], axis=0)
    srcp32 = _pad_edges32(edge_all[:, 0, :])
    dstp32 = _pad_edges32(edge_all[:, 1, :])
    srcp16 = _pad_edges16(edge_all[:, 0, :])
    dstp16 = _pad_edges16(edge_all[:, 1, :])
    wrel_d = (jnp.dot(Wd, jnp.ones((RD,), jnp.float32)) + bd).astype(jnp.float32)
    wrel_t = (jnp.dot(Wt, jnp.ones((RT,), jnp.float32)) + bt).astype(jnp.float32)

    fs_all, fd_all, el128, er128, q0_all, rr_all = _stage_a(
        xpair, w_all, al_all, ar_all, hq_all, hr_all)

    ex32 = _attention(el128.reshape(R * NP, DH), er128.reshape(R * NP, DH),
                      srcp32, dstp32)
    exsplat = _expand(ex32)

    hs1 = _hop(fs_all.reshape(R * H * NP, DH), exsplat, srcp16, dstp16)
    hn1, q1 = _norm(hs1, hq_all)
    hs2 = _hop(hn1.reshape(R * H * NP, DH), exsplat, srcp16, dstp16)
    hn2, q2 = _norm(hs2, hq_all)
    hs3 = _hop(hn2.reshape(R * H * NP, DH), exsplat, srcp16, dstp16)
    hn3, q3 = _norm(hs3, hq_all)

    h_drug = _combine(wrel_d, fd_all, hn1, hn2, hn3,
                      q0_all, q1, q2, q3, rr_all, 0, RD)
    h_target = _combine(wrel_t, fd_all, hn1, hn2, hn3,
                        q0_all, q1, q2, q3, rr_all, RD, RT)
    return (h_drug[:N], h_target[:N])


# speedup vs baseline: 1.6601x; 1.6601x over previous
import jax
import jax.numpy as jnp
from jax import lax
from jax.experimental import pallas as pl
from jax.experimental.pallas import tpu as pltpu
from jax.experimental.pallas import tpu_sc as plsc

N = 10000
D = 128
H = 4
DH = 128
E = 40000
RD = 5
RT = 4
R = RD + RT

NP = 10240
NT = 16
NW = 32
EPW = E // NW
CHA = 128
NCA = 10
EPWP = NCA * CHA
EXR = EPWP // 8
EPT = 2 * EPWP
CHH = 64
NCH = EPT // CHH
SPT = NP // NT
ZB = SPT // 40
BN = 1024
NB = NP // BN
NLANE = 16

_MESH = plsc.VectorSubcoreMesh(core_axis_name="c", subcore_axis_name="s")


def _feat_body(xs_ref, xd_ref, w_ref, al_ref, ar_ref, hq_ref, hr_ref,
               fs_ref, fd_ref, el_ref, er_ref, q0_ref, rr_ref):
    xs = xs_ref[0]
    xd = xd_ref[0]
    w = w_ref[0]
    fs = jnp.dot(xs, w, preferred_element_type=jnp.float32)
    fd = jnp.dot(xd, w, preferred_element_type=jnp.float32)
    els, ers, q0s, rrs = [], [], [], []
    for h in range(H):
        fsh = fs[:, h * DH:(h + 1) * DH]
        fdh = fd[:, h * DH:(h + 1) * DH]
        fs_ref[0, h] = fsh
        fd_ref[0, h] = fdh
        els.append(jnp.sum(fsh * al_ref[0, h][None, :], axis=1))
        ers.append(jnp.sum(fdh * ar_ref[0, h][None, :], axis=1))
        q0s.append(jnp.sum(fdh * hq_ref[0, h][None, :], axis=1))
        rrs.append(jnp.sum(fdh * hr_ref[0, h][None, :], axis=1))
    el4 = jnp.stack(els, axis=-1)
    er4 = jnp.stack(ers, axis=-1)
    z = jnp.zeros((el4.shape[0], DH - 4 * H), jnp.float32)
    el_ref[0] = jnp.concatenate([el4, el4, el4, el4, z], axis=-1)
    er_ref[0] = jnp.concatenate([er4, er4, er4, er4, z], axis=-1)
    q0_ref[0] = jnp.stack(q0s)
    rr_ref[0] = jnp.stack(rrs)


def _stage_a(xpair, w_all, al_all, ar_all, hq_all, hr_all):
    f_sds = jax.ShapeDtypeStruct((R, H, NP, DH), jnp.float32)
    p_sds = jax.ShapeDtypeStruct((R, NP, DH), jnp.float32)
    v_sds = jax.ShapeDtypeStruct((R, H, NP), jnp.float32)
    wmap = lambda r, i: (r, 0, 0)
    return pl.pallas_call(
        _feat_body,
        grid=(R, NB),
        in_specs=[
            pl.BlockSpec((1, BN, D), lambda r, i: (jnp.where(r < RD, 0, 1), i, 0)),
            pl.BlockSpec((1, BN, D), lambda r, i: (jnp.where(r < RD, 1, 0), i, 0)),
            pl.BlockSpec((1, D, H * DH), wmap),
            pl.BlockSpec((1, H, DH), wmap),
            pl.BlockSpec((1, H, DH), wmap),
            pl.BlockSpec((1, H, DH), wmap),
            pl.BlockSpec((1, H, DH), wmap),
        ],
        out_specs=[
            pl.BlockSpec((1, H, BN, DH), lambda r, i: (r, 0, i, 0)),
            pl.BlockSpec((1, H, BN, DH), lambda r, i: (r, 0, i, 0)),
            pl.BlockSpec((1, BN, DH), lambda r, i: (r, i, 0)),
            pl.BlockSpec((1, BN, DH), lambda r, i: (r, i, 0)),
            pl.BlockSpec((1, H, BN), lambda r, i: (r, 0, i)),
            pl.BlockSpec((1, H, BN), lambda r, i: (r, 0, i)),
        ],
        out_shape=[f_sds, f_sds, p_sds, p_sds, v_sds, v_sds],
    )(xpair, xpair, w_all, al_all, ar_all, hq_all, hr_all)


def _att_body(eltab, ertab, srcp, dstp, ex_hbm,
              src_loc, dst_loc, isrc, idst, rs, rd, ex_buf, sem1, sem2):
    c = lax.axis_index("c")
    s = lax.axis_index("s")
    w = 2 * s + c

    def rel_body(r, carry):
        pltpu.sync_copy(srcp.at[r, w], src_loc)
        pltpu.sync_copy(dstp.at[r, w], dst_loc)
        off = r * NP

        def ib(j, ci):
            row = j // 8
            col = (j % 8) * NLANE
            isrc[row, pl.ds(col, NLANE)] = src_loc[row, pl.ds(col, NLANE)] + off
            idst[row, pl.ds(col, NLANE)] = dst_loc[row, pl.ds(col, NLANE)] + off
            return ci
        lax.fori_loop(0, EPWP // NLANE, ib, 0)

        def chunk(ci, cc):
            d1 = pltpu.async_copy(eltab.at[isrc.at[ci]], rs, sem1)
            d2 = pltpu.async_copy(ertab.at[idst.at[ci]], rd, sem2)
            d1.wait()
            d2.wait()

            def per_edge(j, cs):
                q = ci * CHA + j
                ev = rs[j, pl.ds(0, NLANE)] + rd[j, pl.ds(0, NLANE)]
                ev = jnp.maximum(ev, 0.2 * ev)
                ex = jnp.exp(ev)
                ex = jnp.where(q < EPW, ex, 0.0)
                ex_buf[q // 8, pl.ds((q % 8) * NLANE, NLANE)] = ex
                return cs
            lax.fori_loop(0, CHA, per_edge, 0)
            return cc
        lax.fori_loop(0, NCA, chunk, 0)

        pltpu.sync_copy(ex_buf, ex_hbm.at[r, w])
        return carry
    lax.fori_loop(0, R, rel_body, 0)


def _attention(eltab_flat, ertab_flat, srcp32, dstp32):
    fn = pl.kernel(
        _att_body,
        out_type=jax.ShapeDtypeStruct((R, NW, EXR, 128), jnp.float32),
        mesh=_MESH,
        scratch_types=[
            pltpu.VMEM((NCA, CHA), jnp.int32),
            pltpu.VMEM((NCA, CHA), jnp.int32),
            pltpu.VMEM((NCA, CHA), jnp.int32),
            pltpu.VMEM((NCA, CHA), jnp.int32),
            pltpu.VMEM((CHA, DH), jnp.float32),
            pltpu.VMEM((CHA, DH), jnp.float32),
            pltpu.VMEM((EXR, 128), jnp.float32),
            pltpu.SemaphoreType.DMA,
            pltpu.SemaphoreType.DMA,
        ],
    )
    return fn(eltab_flat, ertab_flat, srcp32, dstp32)


def _expand_body(ex_ref, out_ref):
    x = ex_ref[0]
    ys = []
    for h in range(H):
        groups = []
        for g in range(8):
            col = x[:, :, g * NLANE + h:g * NLANE + h + 1]
            groups.append(jnp.broadcast_to(col, (NW, EXR, NLANE)))
        ys.append(jnp.concatenate(groups, axis=-1))
    y = jnp.stack(ys)
    out_ref[...] = y.reshape(H * NW * EXR, 128)


def _expand(ex32):
    return pl.pallas_call(
        _expand_body,
        grid=(R,),
        in_specs=[pl.BlockSpec((1, NW, EXR, 128), lambda r: (r, 0, 0, 0))],
        out_specs=pl.BlockSpec((H * NW * EXR, 128), lambda r: (r, 0)),
        out_shape=jax.ShapeDtypeStruct((R * H * NW * EXR, 128), jnp.float32),
    )(ex32)


def _hop_body(htab, exsp, srcp, dstp, hsum_hbm,
              acc, rows0, rows1, srows, exc0, exc1,
              src_loc, dst_loc, idx_buf, dst64, zbuf,
              gsem0, gsem1, esem0, esem1, ssem, zsem):
    c = lax.axis_index("c")
    s = lax.axis_index("s")
    zero16 = jnp.zeros((NLANE,), jnp.float32)
    rowsv = (rows0, rows1)
    excv = (exc0, exc1)
    gsemv = (gsem0, gsem1)
    esemv = (esem0, esem1)

    def zz(j, cz):
        zbuf[j // 8, pl.ds((j % 8) * NLANE, NLANE)] = zero16
        return cz
    lax.fori_loop(0, ZB * (DH // NLANE), zz, 0)

    def rel_body(r, carry):
        pltpu.sync_copy(srcp.at[r, s], src_loc)
        pltpu.sync_copy(dstp.at[r, s], dst_loc)

        @plsc.parallel_loop(0, EPT // NLANE, 1, unroll=8)
        def db(j):
            row = j // 8
            col = (j % 8) * NLANE
            v = dst_loc[row, pl.ds(col, NLANE)]
            fl = j * NLANE
            dst64[fl // CHH, pl.ds(fl % CHH, NLANE)] = v

        for hh in range(2):
            h = 2 * c + hh
            off = (r * H + h) * NP
            row0 = ((r * H + h) * NW + 2 * s) * EXR

            @plsc.parallel_loop(0, EPT // NLANE, 1, unroll=8)
            def ib(j):
                row = j // 8
                col = (j % 8) * NLANE
                v = src_loc[row, pl.ds(col, NLANE)] + off
                fl = j * NLANE
                idx_buf[fl // CHH, pl.ds(fl % CHH, NLANE)] = v

            def zc(k, cz):
                pltpu.async_copy(zbuf, acc.at[pl.ds(s * SPT + k * ZB, ZB)], zsem)
                return cz
            lax.fori_loop(0, SPT // ZB, zc, 0)

            def zw(k, cz):
                pltpu.make_async_copy(zbuf, acc.at[pl.ds(s * SPT, ZB)], zsem).wait()
                return cz
            lax.fori_loop(0, SPT // ZB, zw, 0)
            plsc.subcore_barrier()

            for b in range(2):
                pltpu.async_copy(htab.at[idx_buf.at[b]], rowsv[b], gsemv[b])
                pltpu.async_copy(
                    exsp.at[pl.ds(row0 + b * (CHH // 8), CHH // 8)],
                    excv[b], esemv[b])

            def pair(g, cc):
                for b in range(2):
                    ci = g * 2 + b
                    pltpu.make_async_copy(
                        htab.at[idx_buf.at[0]], rowsv[b], gsemv[b]).wait()
                    pltpu.make_async_copy(
                        exsp.at[pl.ds(row0, CHH // 8)], excv[b], esemv[b]).wait()

                    @pl.when(ci >= 1)
                    def _drain():
                        pltpu.make_async_copy(
                            srows, acc.at[dst64.at[0]], ssem).wait()

                    @plsc.parallel_loop(0, CHH, 1, unroll=4)
                    def scale(j):
                        av = excv[b][j // 8, pl.ds((j % 8) * NLANE, NLANE)]
                        for v in range(DH // NLANE):
                            srows[j, pl.ds(v * NLANE, NLANE)] = (
                                rowsv[b][j, pl.ds(v * NLANE, NLANE)] * av)
                    pltpu.async_copy(srows, acc.at[dst64.at[ci]],
                                     ssem, add=True)

                    @pl.when(ci + 2 < NCH)
                    def _prefetch():
                        pltpu.async_copy(htab.at[idx_buf.at[ci + 2]],
                                         rowsv[b], gsemv[b])
                        pltpu.async_copy(
                            exsp.at[pl.ds(row0 + (ci + 2) * (CHH // 8), CHH // 8)],
                            excv[b], esemv[b])
                return cc
            lax.fori_loop(0, NCH // 2, pair, 0)
            pltpu.make_async_copy(srows, acc.at[dst64.at[0]], ssem).wait()
            plsc.subcore_barrier()
            pltpu.sync_copy(acc.at[pl.ds(s * SPT, SPT)],
                            hsum_hbm.at[r, h, pl.ds(s * SPT, SPT)])
            plsc.subcore_barrier()
        return carry
    lax.fori_loop(0, R, rel_body, 0)


def _hop(htab_flat, exsplat, srcp16, dstp16):
    fn = pl.kernel(
        _hop_body,
        out_type=jax.ShapeDtypeStruct((R, H, NP, DH), jnp.float32),
        mesh=_MESH,
        scratch_types=[
            pltpu.VMEM_SHARED((NP, DH), jnp.float32),
            pltpu.VMEM((CHH, DH), jnp.float32),
            pltpu.VMEM((CHH, DH), jnp.float32),
            pltpu.VMEM((CHH, DH), jnp.float32),
            pltpu.VMEM((CHH // 8, 128), jnp.float32),
            pltpu.VMEM((CHH // 8, 128), jnp.float32),
            pltpu.VMEM((EPT // CHA, CHA), jnp.int32),
            pltpu.VMEM((EPT // CHA, CHA), jnp.int32),
            pltpu.VMEM((NCH, CHH), jnp.int32),
            pltpu.VMEM((NCH, CHH), jnp.int32),
            pltpu.VMEM((ZB, DH), jnp.float32),
            pltpu.SemaphoreType.DMA,
            pltpu.SemaphoreType.DMA,
            pltpu.SemaphoreType.DMA,
            pltpu.SemaphoreType.DMA,
            pltpu.SemaphoreType.DMA,
            pltpu.SemaphoreType.DMA,
        ],
    )
    return fn(htab_flat, exsplat, srcp16, dstp16)


def _norm_body(hs_ref, hq_ref, hn_ref, q_ref):
    qs = []
    for h in range(H):
        v = hs_ref[0, h]
        nr = jnp.sqrt(jnp.sum(v * v, axis=-1, keepdims=True))
        y = v / (nr + 1e-30)
        hn_ref[0, h] = y
        qs.append(jnp.sum(y * hq_ref[0, h][None, :], axis=-1))
    q_ref[0] = jnp.stack(qs)


def _norm(hsum, hq_all):
    return pl.pallas_call(
        _norm_body,
        grid=(R, NB),
        in_specs=[
            pl.BlockSpec((1, H, BN, DH), lambda r, i: (r, 0, i, 0)),
            pl.BlockSpec((1, H, DH), lambda r, i: (r, 0, 0)),
        ],
        out_specs=[
            pl.BlockSpec((1, H, BN, DH), lambda r, i: (r, 0, i, 0)),
            pl.BlockSpec((1, H, BN), lambda r, i: (r, 0, i)),
        ],
        out_shape=[
            jax.ShapeDtypeStruct((R, H, NP, DH), jnp.float32),
            jax.ShapeDtypeStruct((R, H, NP), jnp.float32),
        ],
    )(hsum, hq_all)


def _comb_body(wrel_ref, fd_ref, h1_ref, h2_ref, h3_ref,
               q0_ref, q1_ref, q2_ref, q3_ref, rr_ref, out_ref):
    r = pl.program_id(1)
    rr = rr_ref[0]

    def leaky(x):
        return jnp.maximum(x, 0.2 * x)
    l0 = leaky(q0_ref[0] + rr)
    l1 = leaky(q1_ref[0] + rr)
    l2 = leaky(q2_ref[0] + rr)
    l3 = leaky(q3_ref[0] + rr)
    m = jnp.maximum(jnp.maximum(l0, l1), jnp.maximum(l2, l3))
    e0 = jnp.exp(l0 - m)
    e1 = jnp.exp(l1 - m)
    e2 = jnp.exp(l2 - m)
    e3 = jnp.exp(l3 - m)
    den = e0 + e1 + e2 + e3
    w0 = e0 / den + 1.0
    w1 = e1 / den
    w2 = e2 / den
    w3 = e3 / den
    acc = jnp.zeros((BN, DH), jnp.float32)
    for h in range(H):
        acc = acc + (w0[h][:, None] * fd_ref[0, h]
                     + w1[h][:, None] * h1_ref[0, h]
                     + w2[h][:, None] * h2_ref[0, h]
                     + w3[h][:, None] * h3_ref[0, h])
    contrib = (wrel_ref[r] * (1.0 / H)) * acc

    @pl.when(r == 0)
    def _init():
        out_ref[...] = contrib

    @pl.when(r > 0)
    def _acc():
        out_ref[...] += contrib


def _combine(wrel, fd_all, h1, h2, h3, q0, q1, q2, q3, rr, off, rg):
    fmap = lambda i, r: (r + off, 0, i, 0)
    vmap_ = lambda i, r: (r + off, 0, i)
    return pl.pallas_call(
        _comb_body,
        grid=(NB, rg),
        in_specs=[
            pl.BlockSpec(memory_space=pltpu.SMEM),
            pl.BlockSpec((1, H, BN, DH), fmap),
            pl.BlockSpec((1, H, BN, DH), fmap),
            pl.BlockSpec((1, H, BN, DH), fmap),
            pl.BlockSpec((1, H, BN, DH), fmap),
            pl.BlockSpec((1, H, BN), vmap_),
            pl.BlockSpec((1, H, BN), vmap_),
            pl.BlockSpec((1, H, BN), vmap_),
            pl.BlockSpec((1, H, BN), vmap_),
            pl.BlockSpec((1, H, BN), vmap_),
        ],
        out_specs=pl.BlockSpec((BN, DH), lambda i, r: (i, 0)),
        out_shape=jax.ShapeDtypeStruct((NP, DH), jnp.float32),
    )(wrel, fd_all, h1, h2, h3, q0, q1, q2, q3, rr)


def _pad_edges32(a):
    a = a.reshape(R, NW, EPW)
    a = jnp.pad(a, ((0, 0), (0, 0), (0, EPWP - EPW)))
    return a.reshape(R, NW, NCA, CHA).astype(jnp.int32)


def _pad_edges16(a):
    a = a.reshape(R, NW, EPW)
    a = jnp.pad(a, ((0, 0), (0, 0), (0, EPWP - EPW)))
    return a.reshape(R, NT, EPT // CHA, CHA).astype(jnp.int32)


def kernel(x_drug, x_target, edge_index_drug, edge_index_target,
           W_drug, attn_l_drug, attn_r_drug, hop_q_drug, hop_r_drug,
           W_target, attn_l_target, attn_r_target, hop_q_target, hop_r_target,
           Wd, bd, Wt, bt):
    pad = ((0, NP - N), (0, 0))
    xpair = jnp.stack([jnp.pad(x_target, pad), jnp.pad(x_drug, pad)])
    w_all = jnp.concatenate([W_drug, W_target], axis=0)
    al_all = jnp.concatenate([attn_l_drug, attn_l_target], axis=0)
    ar_all = jnp.concatenate([attn_r_drug, attn_r_target], axis=0)
    hq_all = jnp.concatenate([hop_q_drug, hop_q_target], axis=0)
    hr_all = jnp.concatenate([hop_r_drug, hop_r_target], axis=0)
    edge_all = jnp.concatenate([edge_index_drug, edge_index_target], axis=0)
    srcp32 = _pad_edges32(edge_all[:, 0, :])
    dstp32 = _pad_edges32(edge_all[:, 1, :])
    srcp16 = _pad_edges16(edge_all[:, 0, :])
    dstp16 = _pad_edges16(edge_all[:, 1, :])
    wrel_d = (jnp.dot(Wd, jnp.ones((RD,), jnp.float32)) + bd).astype(jnp.float32)
    wrel_t = (jnp.dot(Wt, jnp.ones((RT,), jnp.float32)) + bt).astype(jnp.float32)

    fs_all, fd_all, el128, er128, q0_all, rr_all = _stage_a(
        xpair, w_all, al_all, ar_all, hq_all, hr_all)

    ex32 = _attention(el128.reshape(R * NP, DH), er128.reshape(R * NP, DH),
                      srcp32, dstp32)
    exsplat = _expand(ex32)

    hs1 = _hop(fs_all.reshape(R * H * NP, DH), exsplat, srcp16, dstp16)
    hn1, q1 = _norm(hs1, hq_all)
    hs2 = _hop(hn1.reshape(R * H * NP, DH), exsplat, srcp16, dstp16)
    hn2, q2 = _norm(hs2, hq_all)
    hs3 = _hop(hn2.reshape(R * H * NP, DH), exsplat, srcp16, dstp16)
    hn3, q3 = _norm(hs3, hq_all)

    h_drug = _combine(wrel_d, fd_all, hn1, hn2, hn3,
                      q0_all, q1, q2, q3, rr_all, 0, RD)
    h_target = _combine(wrel_t, fd_all, hn1, hn2, hn3,
                        q0_all, q1, q2, q3, rr_all, RD, RT)
    return (h_drug[:N], h_target[:N])


# speedup vs baseline: 1.6862x; 1.0157x over previous
import jax
import jax.numpy as jnp
from jax import lax
from jax.experimental import pallas as pl
from jax.experimental.pallas import tpu as pltpu
from jax.experimental.pallas import tpu_sc as plsc

N = 10000
D = 128
H = 4
DH = 128
E = 40000
RD = 5
RT = 4
R = RD + RT

NP = 10240
NT = 16
NW = 32
EPW = E // NW
CHA = 128
NCA = 10
EPWP = NCA * CHA
EXR = EPWP // 8
EPT = 2 * EPWP
CHH = 64
NCH = EPT // CHH
SPT = NP // NT
ZB = SPT // 40
BN = 1024
NB = NP // BN
NLANE = 16

_MESH = plsc.VectorSubcoreMesh(core_axis_name="c", subcore_axis_name="s")


def _feat_body(xs_ref, xd_ref, w_ref, al_ref, ar_ref, hq_ref, hr_ref,
               fs_ref, fd_ref, el_ref, er_ref, q0_ref, rr_ref):
    xs = xs_ref[0]
    xd = xd_ref[0]
    w = w_ref[0]
    fs = jnp.dot(xs, w, preferred_element_type=jnp.float32)
    fd = jnp.dot(xd, w, preferred_element_type=jnp.float32)
    els, ers, q0s, rrs = [], [], [], []
    for h in range(H):
        fsh = fs[:, h * DH:(h + 1) * DH]
        fdh = fd[:, h * DH:(h + 1) * DH]
        fs_ref[0, h] = fsh
        fd_ref[0, h] = fdh
        els.append(jnp.sum(fsh * al_ref[0, h][None, :], axis=1))
        ers.append(jnp.sum(fdh * ar_ref[0, h][None, :], axis=1))
        q0s.append(jnp.sum(fdh * hq_ref[0, h][None, :], axis=1))
        rrs.append(jnp.sum(fdh * hr_ref[0, h][None, :], axis=1))
    el4 = jnp.stack(els, axis=-1)
    er4 = jnp.stack(ers, axis=-1)
    z = jnp.zeros((el4.shape[0], DH - 4 * H), jnp.float32)
    el_ref[0] = jnp.concatenate([el4, el4, el4, el4, z], axis=-1)
    er_ref[0] = jnp.concatenate([er4, er4, er4, er4, z], axis=-1)
    q0_ref[0] = jnp.stack(q0s)
    rr_ref[0] = jnp.stack(rrs)


def _stage_a(xpair, w_all, al_all, ar_all, hq_all, hr_all):
    f_sds = jax.ShapeDtypeStruct((R, H, NP, DH), jnp.float32)
    p_sds = jax.ShapeDtypeStruct((R, NP, DH), jnp.float32)
    v_sds = jax.ShapeDtypeStruct((R, H, NP), jnp.float32)
    wmap = lambda r, i: (r, 0, 0)
    return pl.pallas_call(
        _feat_body,
        grid=(R, NB),
        in_specs=[
            pl.BlockSpec((1, BN, D), lambda r, i: (jnp.where(r < RD, 0, 1), i, 0)),
            pl.BlockSpec((1, BN, D), lambda r, i: (jnp.where(r < RD, 1, 0), i, 0)),
            pl.BlockSpec((1, D, H * DH), wmap),
            pl.BlockSpec((1, H, DH), wmap),
            pl.BlockSpec((1, H, DH), wmap),
            pl.BlockSpec((1, H, DH), wmap),
            pl.BlockSpec((1, H, DH), wmap),
        ],
        out_specs=[
            pl.BlockSpec((1, H, BN, DH), lambda r, i: (r, 0, i, 0)),
            pl.BlockSpec((1, H, BN, DH), lambda r, i: (r, 0, i, 0)),
            pl.BlockSpec((1, BN, DH), lambda r, i: (r, i, 0)),
            pl.BlockSpec((1, BN, DH), lambda r, i: (r, i, 0)),
            pl.BlockSpec((1, H, BN), lambda r, i: (r, 0, i)),
            pl.BlockSpec((1, H, BN), lambda r, i: (r, 0, i)),
        ],
        out_shape=[f_sds, f_sds, p_sds, p_sds, v_sds, v_sds],
    )(xpair, xpair, w_all, al_all, ar_all, hq_all, hr_all)


def _att_body(eltab, ertab, srcp, dstp, ex_hbm,
              src_loc, dst_loc, isrc, idst, rs0, rs1, rd0, rd1, ex_buf,
              sem1a, sem1b, sem2a, sem2b):
    c = lax.axis_index("c")
    s = lax.axis_index("s")
    w = 2 * s + c
    rsv = (rs0, rs1)
    rdv = (rd0, rd1)
    s1v = (sem1a, sem1b)
    s2v = (sem2a, sem2b)

    def rel_body(r, carry):
        pltpu.sync_copy(srcp.at[r, w], src_loc)
        pltpu.sync_copy(dstp.at[r, w], dst_loc)
        off = r * NP

        @plsc.parallel_loop(0, EPWP // NLANE, 1, unroll=8)
        def ib(j):
            row = j // 8
            col = (j % 8) * NLANE
            isrc[row, pl.ds(col, NLANE)] = src_loc[row, pl.ds(col, NLANE)] + off
            idst[row, pl.ds(col, NLANE)] = dst_loc[row, pl.ds(col, NLANE)] + off

        for b in range(2):
            pltpu.async_copy(eltab.at[isrc.at[b]], rsv[b], s1v[b])
            pltpu.async_copy(ertab.at[idst.at[b]], rdv[b], s2v[b])

        def pair(g, cc):
            for b in range(2):
                ci = g * 2 + b
                pltpu.make_async_copy(eltab.at[isrc.at[0]], rsv[b], s1v[b]).wait()
                pltpu.make_async_copy(ertab.at[idst.at[0]], rdv[b], s2v[b]).wait()

                @plsc.parallel_loop(0, CHA, 1, unroll=4)
                def per_edge(j):
                    q = ci * CHA + j
                    ev = rsv[b][j, pl.ds(0, NLANE)] + rdv[b][j, pl.ds(0, NLANE)]
                    ev = jnp.maximum(ev, 0.2 * ev)
                    ex = jnp.exp(ev)
                    ex = jnp.where(q < EPW, ex, 0.0)
                    ex_buf[q // 8, pl.ds((q % 8) * NLANE, NLANE)] = ex

                @pl.when(ci + 2 < NCA)
                def _prefetch():
                    pltpu.async_copy(eltab.at[isrc.at[ci + 2]], rsv[b], s1v[b])
                    pltpu.async_copy(ertab.at[idst.at[ci + 2]], rdv[b], s2v[b])
            return cc
        lax.fori_loop(0, NCA // 2, pair, 0)

        pltpu.sync_copy(ex_buf, ex_hbm.at[r, w])
        return carry
    lax.fori_loop(0, R, rel_body, 0)


def _attention(eltab_flat, ertab_flat, srcp32, dstp32):
    fn = pl.kernel(
        _att_body,
        out_type=jax.ShapeDtypeStruct((R, NW, EXR, 128), jnp.float32),
        mesh=_MESH,
        scratch_types=[
            pltpu.VMEM((NCA, CHA), jnp.int32),
            pltpu.VMEM((NCA, CHA), jnp.int32),
            pltpu.VMEM((NCA, CHA), jnp.int32),
            pltpu.VMEM((NCA, CHA), jnp.int32),
            pltpu.VMEM((CHA, DH), jnp.float32),
            pltpu.VMEM((CHA, DH), jnp.float32),
            pltpu.VMEM((CHA, DH), jnp.float32),
            pltpu.VMEM((CHA, DH), jnp.float32),
            pltpu.VMEM((EXR, 128), jnp.float32),
            pltpu.SemaphoreType.DMA,
            pltpu.SemaphoreType.DMA,
            pltpu.SemaphoreType.DMA,
            pltpu.SemaphoreType.DMA,
        ],
    )
    return fn(eltab_flat, ertab_flat, srcp32, dstp32)


def _expand_body(ex_ref, out_ref):
    x = ex_ref[0]
    ys = []
    for h in range(H):
        groups = []
        for g in range(8):
            col = x[:, :, g * NLANE + h:g * NLANE + h + 1]
            groups.append(jnp.broadcast_to(col, (NW, EXR, NLANE)))
        ys.append(jnp.concatenate(groups, axis=-1))
    y = jnp.stack(ys)
    out_ref[...] = y.reshape(H * NW * EXR, 128)


def _expand(ex32):
    return pl.pallas_call(
        _expand_body,
        grid=(R,),
        in_specs=[pl.BlockSpec((1, NW, EXR, 128), lambda r: (r, 0, 0, 0))],
        out_specs=pl.BlockSpec((H * NW * EXR, 128), lambda r: (r, 0)),
        out_shape=jax.ShapeDtypeStruct((R * H * NW * EXR, 128), jnp.float32),
    )(ex32)


def _hop_body(htab, exsp, srcp, dstp, hsum_hbm,
              acc, rows0, rows1, srows, exc0, exc1,
              src_loc, dst_loc, idx_buf, dst64, zbuf,
              gsem0, gsem1, esem0, esem1, ssem, zsem):
    c = lax.axis_index("c")
    s = lax.axis_index("s")
    zero16 = jnp.zeros((NLANE,), jnp.float32)
    rowsv = (rows0, rows1)
    excv = (exc0, exc1)
    gsemv = (gsem0, gsem1)
    esemv = (esem0, esem1)

    def zz(j, cz):
        zbuf[j // 8, pl.ds((j % 8) * NLANE, NLANE)] = zero16
        return cz
    lax.fori_loop(0, ZB * (DH // NLANE), zz, 0)

    def rel_body(r, carry):
        pltpu.sync_copy(srcp.at[r, s], src_loc)
        pltpu.sync_copy(dstp.at[r, s], dst_loc)

        @plsc.parallel_loop(0, EPT // NLANE, 1, unroll=8)
        def db(j):
            row = j // 8
            col = (j % 8) * NLANE
            v = dst_loc[row, pl.ds(col, NLANE)]
            fl = j * NLANE
            dst64[fl // CHH, pl.ds(fl % CHH, NLANE)] = v

        for hh in range(2):
            h = 2 * c + hh
            off = (r * H + h) * NP
            row0 = ((r * H + h) * NW + 2 * s) * EXR

            @plsc.parallel_loop(0, EPT // NLANE, 1, unroll=8)
            def ib(j):
                row = j // 8
                col = (j % 8) * NLANE
                v = src_loc[row, pl.ds(col, NLANE)] + off
                fl = j * NLANE
                idx_buf[fl // CHH, pl.ds(fl % CHH, NLANE)] = v

            def zc(k, cz):
                pltpu.async_copy(zbuf, acc.at[pl.ds(s * SPT + k * ZB, ZB)], zsem)
                return cz
            lax.fori_loop(0, SPT // ZB, zc, 0)

            def zw(k, cz):
                pltpu.make_async_copy(zbuf, acc.at[pl.ds(s * SPT, ZB)], zsem).wait()
                return cz
            lax.fori_loop(0, SPT // ZB, zw, 0)
            plsc.subcore_barrier()

            for b in range(2):
                pltpu.async_copy(htab.at[idx_buf.at[b]], rowsv[b], gsemv[b])
                pltpu.async_copy(
                    exsp.at[pl.ds(row0 + b * (CHH // 8), CHH // 8)],
                    excv[b], esemv[b])

            def pair(g, cc):
                for b in range(2):
                    ci = g * 2 + b
                    pltpu.make_async_copy(
                        htab.at[idx_buf.at[0]], rowsv[b], gsemv[b]).wait()
                    pltpu.make_async_copy(
                        exsp.at[pl.ds(row0, CHH // 8)], excv[b], esemv[b]).wait()

                    @pl.when(ci >= 1)
                    def _drain():
                        pltpu.make_async_copy(
                            srows, acc.at[dst64.at[0]], ssem).wait()

                    @plsc.parallel_loop(0, CHH, 1, unroll=4)
                    def scale(j):
                        av = excv[b][j // 8, pl.ds((j % 8) * NLANE, NLANE)]
                        for v in range(DH // NLANE):
                            srows[j, pl.ds(v * NLANE, NLANE)] = (
                                rowsv[b][j, pl.ds(v * NLANE, NLANE)] * av)
                    pltpu.async_copy(srows, acc.at[dst64.at[ci]],
                                     ssem, add=True)

                    @pl.when(ci + 2 < NCH)
                    def _prefetch():
                        pltpu.async_copy(htab.at[idx_buf.at[ci + 2]],
                                         rowsv[b], gsemv[b])
                        pltpu.async_copy(
                            exsp.at[pl.ds(row0 + (ci + 2) * (CHH // 8), CHH // 8)],
                            excv[b], esemv[b])
                return cc
            lax.fori_loop(0, NCH // 2, pair, 0)
            pltpu.make_async_copy(srows, acc.at[dst64.at[0]], ssem).wait()
            plsc.subcore_barrier()
            pltpu.sync_copy(acc.at[pl.ds(s * SPT, SPT)],
                            hsum_hbm.at[r, h, pl.ds(s * SPT, SPT)])
            plsc.subcore_barrier()
        return carry
    lax.fori_loop(0, R, rel_body, 0)


def _hop(htab_flat, exsplat, srcp16, dstp16):
    fn = pl.kernel(
        _hop_body,
        out_type=jax.ShapeDtypeStruct((R, H, NP, DH), jnp.float32),
        mesh=_MESH,
        scratch_types=[
            pltpu.VMEM_SHARED((NP, DH), jnp.float32),
            pltpu.VMEM((CHH, DH), jnp.float32),
            pltpu.VMEM((CHH, DH), jnp.float32),
            pltpu.VMEM((CHH, DH), jnp.float32),
            pltpu.VMEM((CHH // 8, 128), jnp.float32),
            pltpu.VMEM((CHH // 8, 128), jnp.float32),
            pltpu.VMEM((EPT // CHA, CHA), jnp.int32),
            pltpu.VMEM((EPT // CHA, CHA), jnp.int32),
            pltpu.VMEM((NCH, CHH), jnp.int32),
            pltpu.VMEM((NCH, CHH), jnp.int32),
            pltpu.VMEM((ZB, DH), jnp.float32),
            pltpu.SemaphoreType.DMA,
            pltpu.SemaphoreType.DMA,
            pltpu.SemaphoreType.DMA,
            pltpu.SemaphoreType.DMA,
            pltpu.SemaphoreType.DMA,
            pltpu.SemaphoreType.DMA,
        ],
    )
    return fn(htab_flat, exsplat, srcp16, dstp16)


def _norm_body(hs_ref, hq_ref, hn_ref, q_ref):
    qs = []
    for h in range(H):
        v = hs_ref[0, h]
        nr = jnp.sqrt(jnp.sum(v * v, axis=-1, keepdims=True))
        y = v / (nr + 1e-30)
        hn_ref[0, h] = y
        qs.append(jnp.sum(y * hq_ref[0, h][None, :], axis=-1))
    q_ref[0] = jnp.stack(qs)


def _norm(hsum, hq_all):
    return pl.pallas_call(
        _norm_body,
        grid=(R, NB),
        in_specs=[
            pl.BlockSpec((1, H, BN, DH), lambda r, i: (r, 0, i, 0)),
            pl.BlockSpec((1, H, DH), lambda r, i: (r, 0, 0)),
        ],
        out_specs=[
            pl.BlockSpec((1, H, BN, DH), lambda r, i: (r, 0, i, 0)),
            pl.BlockSpec((1, H, BN), lambda r, i: (r, 0, i)),
        ],
        out_shape=[
            jax.ShapeDtypeStruct((R, H, NP, DH), jnp.float32),
            jax.ShapeDtypeStruct((R, H, NP), jnp.float32),
        ],
    )(hsum, hq_all)


def _comb_body(wrel_ref, fd_ref, h1_ref, h2_ref, h3_ref,
               q0_ref, q1_ref, q2_ref, q3_ref, rr_ref, out_ref):
    r = pl.program_id(1)
    rr = rr_ref[0]

    def leaky(x):
        return jnp.maximum(x, 0.2 * x)
    l0 = leaky(q0_ref[0] + rr)
    l1 = leaky(q1_ref[0] + rr)
    l2 = leaky(q2_ref[0] + rr)
    l3 = leaky(q3_ref[0] + rr)
    m = jnp.maximum(jnp.maximum(l0, l1), jnp.maximum(l2, l3))
    e0 = jnp.exp(l0 - m)
    e1 = jnp.exp(l1 - m)
    e2 = jnp.exp(l2 - m)
    e3 = jnp.exp(l3 - m)
    den = e0 + e1 + e2 + e3
    w0 = e0 / den + 1.0
    w1 = e1 / den
    w2 = e2 / den
    w3 = e3 / den
    acc = jnp.zeros((BN, DH), jnp.float32)
    for h in range(H):
        acc = acc + (w0[h][:, None] * fd_ref[0, h]
                     + w1[h][:, None] * h1_ref[0, h]
                     + w2[h][:, None] * h2_ref[0, h]
                     + w3[h][:, None] * h3_ref[0, h])
    contrib = (wrel_ref[r] * (1.0 / H)) * acc

    @pl.when(r == 0)
    def _init():
        out_ref[...] = contrib

    @pl.when(r > 0)
    def _acc():
        out_ref[...] += contrib


def _combine(wrel, fd_all, h1, h2, h3, q0, q1, q2, q3, rr, off, rg):
    fmap = lambda i, r: (r + off, 0, i, 0)
    vmap_ = lambda i, r: (r + off, 0, i)
    return pl.pallas_call(
        _comb_body,
        grid=(NB, rg),
        in_specs=[
            pl.BlockSpec(memory_space=pltpu.SMEM),
            pl.BlockSpec((1, H, BN, DH), fmap),
            pl.BlockSpec((1, H, BN, DH), fmap),
            pl.BlockSpec((1, H, BN, DH), fmap),
            pl.BlockSpec((1, H, BN, DH), fmap),
            pl.BlockSpec((1, H, BN), vmap_),
            pl.BlockSpec((1, H, BN), vmap_),
            pl.BlockSpec((1, H, BN), vmap_),
            pl.BlockSpec((1, H, BN), vmap_),
            pl.BlockSpec((1, H, BN), vmap_),
        ],
        out_specs=pl.BlockSpec((BN, DH), lambda i, r: (i, 0)),
        out_shape=jax.ShapeDtypeStruct((NP, DH), jnp.float32),
    )(wrel, fd_all, h1, h2, h3, q0, q1, q2, q3, rr)


def _pad_edges32(a):
    a = a.reshape(R, NW, EPW)
    a = jnp.pad(a, ((0, 0), (0, 0), (0, EPWP - EPW)))
    return a.reshape(R, NW, NCA, CHA).astype(jnp.int32)


def _pad_edges16(a):
    a = a.reshape(R, NW, EPW)
    a = jnp.pad(a, ((0, 0), (0, 0), (0, EPWP - EPW)))
    return a.reshape(R, NT, EPT // CHA, CHA).astype(jnp.int32)


def kernel(x_drug, x_target, edge_index_drug, edge_index_target,
           W_drug, attn_l_drug, attn_r_drug, hop_q_drug, hop_r_drug,
           W_target, attn_l_target, attn_r_target, hop_q_target, hop_r_target,
           Wd, bd, Wt, bt):
    pad = ((0, NP - N), (0, 0))
    xpair = jnp.stack([jnp.pad(x_target, pad), jnp.pad(x_drug, pad)])
    w_all = jnp.concatenate([W_drug, W_target], axis=0)
    al_all = jnp.concatenate([attn_l_drug, attn_l_target], axis=0)
    ar_all = jnp.concatenate([attn_r_drug, attn_r_target], axis=0)
    hq_all = jnp.concatenate([hop_q_drug, hop_q_target], axis=0)
    hr_all = jnp.concatenate([hop_r_drug, hop_r_target], axis=0)
    edge_all = jnp.concatenate([edge_index_drug, edge_index_target], axis=0)
    srcp32 = _pad_edges32(edge_all[:, 0, :])
    dstp32 = _pad_edges32(edge_all[:, 1, :])
    srcp16 = _pad_edges16(edge_all[:, 0, :])
    dstp16 = _pad_edges16(edge_all[:, 1, :])
    wrel_d = (jnp.dot(Wd, jnp.ones((RD,), jnp.float32)) + bd).astype(jnp.float32)
    wrel_t = (jnp.dot(Wt, jnp.ones((RT,), jnp.float32)) + bt).astype(jnp.float32)

    fs_all, fd_all, el128, er128, q0_all, rr_all = _stage_a(
        xpair, w_all, al_all, ar_all, hq_all, hr_all)

    ex32 = _attention(el128.reshape(R * NP, DH), er128.reshape(R * NP, DH),
                      srcp32, dstp32)
    exsplat = _expand(ex32)

    hs1 = _hop(fs_all.reshape(R * H * NP, DH), exsplat, srcp16, dstp16)
    hn1, q1 = _norm(hs1, hq_all)
    hs2 = _hop(hn1.reshape(R * H * NP, DH), exsplat, srcp16, dstp16)
    hn2, q2 = _norm(hs2, hq_all)
    hs3 = _hop(hn2.reshape(R * H * NP, DH), exsplat, srcp16, dstp16)
    hn3, q3 = _norm(hs3, hq_all)

    h_drug = _combine(wrel_d, fd_all, hn1, hn2, hn3,
                      q0_all, q1, q2, q3, rr_all, 0, RD)
    h_target = _combine(wrel_t, fd_all, hn1, hn2, hn3,
                        q0_all, q1, q2, q3, rr_all, RD, RT)
    return (h_drug[:N], h_target[:N])
